# baseline (speedup 1.0000x reference)
import functools

import jax
import jax.numpy as jnp
from jax import lax
from jax.experimental import pallas as pl
from jax.experimental.pallas import tpu as pltpu


def _maxmatch_body(b_ref, a_ref, max_ref, idx_ref, *, TD, ND):
    j = pl.program_id(2)
    a = a_ref[0]
    b = b_ref[0]
    s = lax.dot_general(b, a, (((1,), (1,)), ((), ())))
    tmax = jnp.max(s, axis=0, keepdims=True)
    dio = lax.broadcasted_iota(jnp.int32, s.shape, 0) + j * TD
    targ = jnp.min(jnp.where(s == tmax, dio, ND * TD), axis=0, keepdims=True)

    @pl.when(j == 0)
    def _():
        max_ref[0] = tmax
        idx_ref[0] = targ

    @pl.when(j > 0)
    def _():
        old = max_ref[0]
        upd = tmax > old
        max_ref[0] = jnp.where(upd, tmax, old)
        idx_ref[0] = jnp.where(upd, targ, idx_ref[0])


def _node_max_idx(a, b, TS=512, TD=512):
    B, T, K = a.shape
    NS, ND = T // TS, T // TD
    grid = (B, NS, ND)
    out = pl.pallas_call(
        functools.partial(_maxmatch_body, TD=TD, ND=ND),
        grid=grid,
        in_specs=[
            pl.BlockSpec((1, TD, K), lambda bi, si, di: (bi * ND + di, 0, 0)),
            pl.BlockSpec((1, TS, K), lambda bi, si, di: (bi * NS + si, 0, 0)),
        ],
        out_specs=[
            pl.BlockSpec((1, 1, TS), lambda bi, si, di: (bi * NS + si, 0, 0)),
            pl.BlockSpec((1, 1, TS), lambda bi, si, di: (bi * NS + si, 0, 0)),
        ],
        out_shape=[
            jax.ShapeDtypeStruct((B * NS, 1, TS), jnp.float32),
            jax.ShapeDtypeStruct((B * NS, 1, TS), jnp.int32),
        ],
    )(b.reshape(B * ND, TD, K), a.reshape(B * NS, TS, K))
    node_max = out[0].reshape(B, T)
    node_idx = out[1].reshape(B, T)
    return node_max, node_idx


def _rank_body(vj_ref, vi_ref, rank_ref, *, TI, TJ):
    jb = pl.program_id(2)
    vj = vj_ref[0]
    vi = vi_ref[0]
    jid = lax.broadcasted_iota(jnp.int32, (TJ, TI), 0) + jb * TJ
    iid = lax.broadcasted_iota(jnp.int32, (TJ, TI), 1) + pl.program_id(1) * TI
    cmp = (vj > vi) | ((vj == vi) & (jid < iid))
    cnt = jnp.sum(cmp.astype(jnp.int32), axis=0, keepdims=True)

    @pl.when(jb == 0)
    def _():
        rank_ref[0] = cnt

    @pl.when(jb > 0)
    def _():
        rank_ref[0] = rank_ref[0] + cnt


def _stable_rank(node_max, TI=512, TJ=512):
    B, T = node_max.shape
    NI, NJ = T // TI, T // TJ
    grid = (B, NI, NJ)
    rank = pl.pallas_call(
        functools.partial(_rank_body, TI=TI, TJ=TJ),
        grid=grid,
        in_specs=[
            pl.BlockSpec((1, TJ, 1), lambda bi, ii, ji: (bi * NJ + ji, 0, 0)),
            pl.BlockSpec((1, 1, TI), lambda bi, ii, ji: (bi * NI + ii, 0, 0)),
        ],
        out_specs=pl.BlockSpec((1, 1, TI), lambda bi, ii, ji: (bi * NI + ii, 0, 0)),
        out_shape=jax.ShapeDtypeStruct((B * NI, 1, TI), jnp.int32),
    )(node_max.reshape(B * NJ, TJ, 1), node_max.reshape(B * NI, 1, TI))
    return rank.reshape(B, T)


def kernel(x, metric):
    B, N, C = x.shape
    T = N // 2
    r = min(2048, T)

    m = metric / jnp.linalg.norm(metric, axis=-1, keepdims=True)
    a, b = m[..., ::2, :], m[..., 1::2, :]

    node_max, node_idx = _node_max_idx(a, b)
    rank = _stable_rank(node_max)

    ii = jnp.arange(T, dtype=jnp.int32)
    bi = jnp.arange(B)[:, None]
    inv = jnp.zeros((B, T), jnp.int32).at[bi, rank].set(jnp.broadcast_to(ii[None], (B, T)))

    unm_idx, src_idx = inv[:, r:], inv[:, :r]
    dst_idx = jnp.take_along_axis(node_idx, src_idx, axis=-1)

    x4 = x.reshape(B, T, 2, C)
    xsrc, xdst = x4[:, :, 0], x4[:, :, 1]
    unm = jnp.take_along_axis(xsrc, unm_idx[..., None], axis=1)
    srcg = jnp.take_along_axis(xsrc, src_idx[..., None], axis=1)
    acc = xdst.at[bi, dst_idx].add(srcg)
    cnt = jnp.ones((B, T), x.dtype).at[bi, dst_idx].add(1.0)
    return jnp.concatenate([unm, acc / cnt[..., None]], axis=1)


# speedup vs baseline: 1.1325x; 1.1325x over previous
import functools

import jax
import jax.numpy as jnp
from jax import lax
from jax.experimental import pallas as pl
from jax.experimental.pallas import tpu as pltpu
from jax.experimental.pallas import tpu_sc as plsc


def _maxmatch_body(b_ref, a_ref, max_ref, idx_ref, *, TD, ND):
    j = pl.program_id(2)
    a = a_ref[0]
    b = b_ref[0]
    s = lax.dot_general(b, a, (((1,), (1,)), ((), ())))
    tmax = jnp.max(s, axis=0, keepdims=True)
    dio = lax.broadcasted_iota(jnp.int32, s.shape, 0) + j * TD
    targ = jnp.min(jnp.where(s == tmax, dio, ND * TD), axis=0, keepdims=True)

    @pl.when(j == 0)
    def _():
        max_ref[0] = tmax
        idx_ref[0] = targ

    @pl.when(j > 0)
    def _():
        old = max_ref[0]
        upd = tmax > old
        max_ref[0] = jnp.where(upd, tmax, old)
        idx_ref[0] = jnp.where(upd, targ, idx_ref[0])


def _node_max_idx(a, b, TS=512, TD=512):
    B, T, K = a.shape
    NS, ND = T // TS, T // TD
    grid = (B, NS, ND)
    out = pl.pallas_call(
        functools.partial(_maxmatch_body, TD=TD, ND=ND),
        grid=grid,
        in_specs=[
            pl.BlockSpec((1, TD, K), lambda bi, si, di: (bi * ND + di, 0, 0)),
            pl.BlockSpec((1, TS, K), lambda bi, si, di: (bi * NS + si, 0, 0)),
        ],
        out_specs=[
            pl.BlockSpec((1, 1, TS), lambda bi, si, di: (bi * NS + si, 0, 0)),
            pl.BlockSpec((1, 1, TS), lambda bi, si, di: (bi * NS + si, 0, 0)),
        ],
        out_shape=[
            jax.ShapeDtypeStruct((B * NS, 1, TS), jnp.float32),
            jax.ShapeDtypeStruct((B * NS, 1, TS), jnp.int32),
        ],
    )(b.reshape(B * ND, TD, K), a.reshape(B * NS, TS, K))
    node_max = out[0].reshape(B, T)
    node_idx = out[1].reshape(B, T)
    return node_max, node_idx


def _rank_body(vj_ref, vi_ref, rank_ref, *, TI, TJ):
    jb = pl.program_id(2)
    vj = vj_ref[0]
    vi = vi_ref[0]
    jid = lax.broadcasted_iota(jnp.int32, (TJ, TI), 0) + jb * TJ
    iid = lax.broadcasted_iota(jnp.int32, (TJ, TI), 1) + pl.program_id(1) * TI
    cmp = (vj > vi) | ((vj == vi) & (jid < iid))
    cnt = jnp.sum(cmp.astype(jnp.int32), axis=0, keepdims=True)

    @pl.when(jb == 0)
    def _():
        rank_ref[0] = cnt

    @pl.when(jb > 0)
    def _():
        rank_ref[0] = rank_ref[0] + cnt


def _stable_rank(node_max, TI=512, TJ=512):
    B, T = node_max.shape
    NI, NJ = T // TI, T // TJ
    grid = (B, NI, NJ)
    rank = pl.pallas_call(
        functools.partial(_rank_body, TI=TI, TJ=TJ),
        grid=grid,
        in_specs=[
            pl.BlockSpec((1, TJ, 1), lambda bi, ii, ji: (bi * NJ + ji, 0, 0)),
            pl.BlockSpec((1, 1, TI), lambda bi, ii, ji: (bi * NI + ii, 0, 0)),
        ],
        out_specs=pl.BlockSpec((1, 1, TI), lambda bi, ii, ji: (bi * NI + ii, 0, 0)),
        out_shape=jax.ShapeDtypeStruct((B * NI, 1, TI), jnp.int32),
    )(node_max.reshape(B * NJ, TJ, 1), node_max.reshape(B * NI, 1, TI))
    return rank.reshape(B, T)


def _invdval_body(rk_ref, nd_ref, inv_ref, dval_ref, *, TI, TP):
    ib = pl.program_id(2)
    rk = rk_ref[0]
    nd = nd_ref[0]
    pids = lax.broadcasted_iota(jnp.int32, (TI, TP), 1) + pl.program_id(1) * TP
    iids = lax.broadcasted_iota(jnp.int32, (TI, TP), 0) + ib * TI
    eq = (rk == pids).astype(jnp.int32)
    inv_c = jnp.sum(eq * iids, axis=0, keepdims=True)
    dval_c = jnp.sum(eq * nd, axis=0, keepdims=True)

    @pl.when(ib == 0)
    def _():
        inv_ref[0] = inv_c
        dval_ref[0] = dval_c

    @pl.when(ib > 0)
    def _():
        inv_ref[0] = inv_ref[0] + inv_c
        dval_ref[0] = dval_ref[0] + dval_c


def _inv_dval(rank, node_idx, TI=512, TP=512):
    B, T = rank.shape
    NI, NP = T // TI, T // TP
    grid = (B, NP, NI)
    outs = pl.pallas_call(
        functools.partial(_invdval_body, TI=TI, TP=TP),
        grid=grid,
        in_specs=[
            pl.BlockSpec((1, TI, 1), lambda bi, pi, ii: (bi * NI + ii, 0, 0)),
            pl.BlockSpec((1, TI, 1), lambda bi, pi, ii: (bi * NI + ii, 0, 0)),
        ],
        out_specs=[
            pl.BlockSpec((1, 1, TP), lambda bi, pi, ii: (bi * NP + pi, 0, 0)),
            pl.BlockSpec((1, 1, TP), lambda bi, pi, ii: (bi * NP + pi, 0, 0)),
        ],
        out_shape=[
            jax.ShapeDtypeStruct((B * NP, 1, TP), jnp.int32),
            jax.ShapeDtypeStruct((B * NP, 1, TP), jnp.int32),
        ],
    )(rank.reshape(B * NI, TI, 1), node_idx.reshape(B * NI, TI, 1))
    return outs[0].reshape(B, T), outs[1].reshape(B, T)


_NC, _NS = 2, 16


def _gather_sc_body(xf, inv_h, outbuf, srcg, inv64, idx64, rows, *, B, N, T, r):
    c = lax.axis_index("c")
    s = lax.axis_index("s")
    w = c * _NS + s
    i16 = lax.broadcasted_iota(jnp.int32, (16,), 0)
    UNM = T - r
    upw = UNM // (_NC * _NS)
    p0 = w * upw

    for b in range(B):
        pltpu.sync_copy(inv_h.at[b, pl.ds(r + p0, upw)], inv64)
        for g in range(upw // 16):
            iv = plsc.load_gather(inv64, [i16 + g * 16])
            idx64[pl.ds(g * 16, 16)] = b * N + iv * 2
        pltpu.sync_copy(xf.at[idx64], rows)
        pltpu.sync_copy(rows, outbuf.at[pl.ds(b * (T + r) + p0, upw)])
        pltpu.sync_copy(inv_h.at[b, pl.ds(p0, upw)], inv64)
        for g in range(upw // 16):
            iv = plsc.load_gather(inv64, [i16 + g * 16])
            idx64[pl.ds(g * 16, 16)] = b * N + iv * 2
        pltpu.sync_copy(xf.at[idx64], rows)
        pltpu.sync_copy(rows, srcg.at[pl.ds(b * r + p0, upw)])


def _gather_sc(x, inv, r):
    B, N, C = x.shape
    T = N // 2
    xf = x.reshape(B * N, C)
    mesh = plsc.VectorSubcoreMesh(core_axis_name="c", subcore_axis_name="s")
    outbuf, srcg = pl.kernel(
        functools.partial(_gather_sc_body, B=B, N=N, T=T, r=r),
        out_type=[
            jax.ShapeDtypeStruct((B * (N - r), C), jnp.float32),
            jax.ShapeDtypeStruct((B * r, C), jnp.float32),
        ],
        mesh=mesh,
        compiler_params=pltpu.CompilerParams(needs_layout_passes=False),
        scratch_types=[
            pltpu.VMEM((64,), jnp.int32),
            pltpu.VMEM((64,), jnp.int32),
            pltpu.VMEM((64, 1024), jnp.float32),
        ],
    )(xf, inv)
    return outbuf, srcg


def _segsum_body(dv_ref, srcg_ref, x4_ref, _obin_ref, out_ref, cnt_ref,
                 *, TD, TQ, NQ):
    q = pl.program_id(2)
    dv = dv_ref[0]
    sg = srcg_ref[0]
    dids = lax.broadcasted_iota(jnp.int32, (TD, TQ), 0) + pl.program_id(1) * TD
    S = (dv == dids).astype(jnp.float32)
    acc = lax.dot_general(S, sg, (((1,), (0,)), ((), ())))
    cnt = jnp.broadcast_to(jnp.sum(S, axis=1, keepdims=True), (TD, 128))

    @pl.when(q == 0)
    def _():
        out_ref[0] = acc
        cnt_ref[...] = cnt

    @pl.when(q > 0)
    def _():
        out_ref[0] = out_ref[0] + acc
        cnt_ref[...] = cnt_ref[...] + cnt

    @pl.when(q == NQ - 1)
    def _():
        C = out_ref.shape[-1]
        xdst = x4_ref[0][:, C:]
        ss = cnt_ref[:, 0:1] + 1.0
        out_ref[0] = (out_ref[0] + xdst) / ss


def _segsum(dval2, srcg, x, outbuf, r, TD=512, TQ=512):
    B, N, C = x.shape
    T = N // 2
    ND, NQ = T // TD, r // TQ
    grid = (B, ND, NQ)
    DOFF = (T - r + TD - 1) // TD
    out = pl.pallas_call(
        functools.partial(_segsum_body, TD=TD, TQ=TQ, NQ=NQ),
        grid=grid,
        in_specs=[
            pl.BlockSpec((1, 1, TQ), lambda bi, di, qi: (bi * NQ + qi, 0, 0)),
            pl.BlockSpec((1, TQ, C), lambda bi, di, qi: (bi * NQ + qi, 0, 0)),
            pl.BlockSpec((1, TD, 2 * C), lambda bi, di, qi: (bi, di, 0)),
            pl.BlockSpec(memory_space=pltpu.MemorySpace.HBM),
        ],
        out_specs=pl.BlockSpec((1, TD, C), lambda bi, di, qi: (bi, di + DOFF, 0)),
        out_shape=jax.ShapeDtypeStruct((B, N - r, C), jnp.float32),
        scratch_shapes=[pltpu.VMEM((TD, 128), jnp.float32)],
        input_output_aliases={3: 0},
    )(
        dval2.reshape(B * NQ, 1, TQ),
        srcg.reshape(B * NQ, TQ, C),
        x.reshape(B, T, 2 * C),
        outbuf.reshape(B, N - r, C),
    )
    return out


def kernel(x, metric):
    B, N, C = x.shape
    T = N // 2
    r = min(2048, T)

    m = metric / jnp.linalg.norm(metric, axis=-1, keepdims=True)
    a, b = m[..., ::2, :], m[..., 1::2, :]

    node_max, node_idx = _node_max_idx(a, b)
    rank = _stable_rank(node_max)
    inv, dval = _inv_dval(rank, node_idx)
    outbuf, srcg = _gather_sc(x, inv, r)
    return _segsum(dval[:, :r], srcg, x, outbuf, r)
